# baseline (speedup 1.0000x reference)
import functools

import jax
import jax.numpy as jnp
from jax import lax
from jax.experimental import pallas as pl
from jax.experimental.pallas import tpu as pltpu
from jax.experimental.pallas import tpu_sc as plsc

VOCAB = 1000000
DIM = 64
BATCH = 16384
HIST = 200

_NC = 2
_NS = 16
_NW = _NC * _NS

_BT = BATCH * HIST
_B_PER_W = _BT // _NW
_CHUNK = 512
_STEPS = _B_PER_W // _CHUNK


def _make_gather():
    mesh = plsc.VectorSubcoreMesh(core_axis_name="c", subcore_axis_name="s")

    @functools.partial(
        pl.kernel,
        mesh=mesh,
        out_type=jax.ShapeDtypeStruct((_BT, DIM), jnp.float32),
        scratch_types=[
            pltpu.VMEM((_CHUNK,), jnp.int32),
            pltpu.VMEM((_CHUNK, DIM), jnp.float32),
            pltpu.SemaphoreType.DMA,
        ],
        compiler_params=pltpu.CompilerParams(use_tc_tiling_on_sc=False),
    )
    def gather_kernel(idx_hbm, table_hbm, out_hbm, idx_v, rows_v, sem):
        wid = lax.axis_index("s") * _NC + lax.axis_index("c")
        base = wid * _B_PER_W

        def step(i, carry):
            off = base + i * _CHUNK
            pltpu.sync_copy(idx_hbm.at[pl.ds(off, _CHUNK)], idx_v)
            pltpu.async_copy(table_hbm.at[idx_v], rows_v, sem).wait()
            pltpu.sync_copy(rows_v, out_hbm.at[pl.ds(off, _CHUNK)])
            return carry

        lax.fori_loop(0, _STEPS, step, 0)

    return gather_kernel


_gather = _make_gather()


@jax.jit
def kernel(x, table):
    flat = _gather(x.reshape(_BT).astype(jnp.int32), table)
    return flat.reshape(BATCH, HIST, DIM)


# speedup vs baseline: 1.0542x; 1.0542x over previous
import functools

import jax
import jax.numpy as jnp
from jax import lax
from jax.experimental import pallas as pl
from jax.experimental.pallas import tpu as pltpu
from jax.experimental.pallas import tpu_sc as plsc

VOCAB = 1000000
DIM = 64
BATCH = 16384
HIST = 200

_NC = 2
_NS = 16
_NW = _NC * _NS

_BT = BATCH * HIST
_B_PER_W = _BT // _NW
_CHUNK = 512
_STEPS = _B_PER_W // _CHUNK
_PAIRS = _STEPS // 2


def _make_gather():
    mesh = plsc.VectorSubcoreMesh(core_axis_name="c", subcore_axis_name="s")

    @functools.partial(
        pl.kernel,
        mesh=mesh,
        out_type=jax.ShapeDtypeStruct((_BT, DIM), jnp.float32),
        scratch_types=[
            pltpu.VMEM((_CHUNK,), jnp.int32),
            pltpu.VMEM((_CHUNK,), jnp.int32),
            pltpu.VMEM((_CHUNK, DIM), jnp.float32),
            pltpu.VMEM((_CHUNK, DIM), jnp.float32),
            pltpu.SemaphoreType.DMA,
            pltpu.SemaphoreType.DMA,
            pltpu.SemaphoreType.DMA,
            pltpu.SemaphoreType.DMA,
        ],
        compiler_params=pltpu.CompilerParams(use_tc_tiling_on_sc=False),
    )
    def gather_kernel(idx_hbm, table_hbm, out_hbm,
                      idx0, idx1, rows0, rows1,
                      semg0, semg1, sems0, sems1):
        wid = lax.axis_index("s") * _NC + lax.axis_index("c")
        base = wid * _B_PER_W
        idx = (idx0, idx1)
        rows = (rows0, rows1)
        semg = (semg0, semg1)
        sems = (sems0, sems1)

        def issue_gather(c, b):
            off = base + c * _CHUNK
            pltpu.sync_copy(idx_hbm.at[pl.ds(off, _CHUNK)], idx[b])
            pltpu.async_copy(table_hbm.at[idx[b]], rows[b], semg[b])

        issue_gather(0, 0)
        issue_gather(1, 1)

        def pair(p, carry):
            for b in range(2):
                c = 2 * p + b
                off = base + c * _CHUNK
                pltpu.make_async_copy(table_hbm.at[idx[b]], rows[b],
                                      semg[b]).wait()
                pltpu.async_copy(rows[b], out_hbm.at[pl.ds(off, _CHUNK)],
                                 sems[b])

                @pl.when(p < _PAIRS - 1)
                def _():
                    pltpu.make_async_copy(
                        rows[b], out_hbm.at[pl.ds(off, _CHUNK)],
                        sems[b]).wait()
                    issue_gather(c + 2, b)

            return carry

        lax.fori_loop(0, _PAIRS, pair, 0)

        off_last = base + (_STEPS - 2) * _CHUNK
        pltpu.make_async_copy(rows[0], out_hbm.at[pl.ds(off_last, _CHUNK)],
                              sems[0]).wait()
        pltpu.make_async_copy(rows[1],
                              out_hbm.at[pl.ds(off_last + _CHUNK, _CHUNK)],
                              sems[1]).wait()

    return gather_kernel


_gather = _make_gather()


@jax.jit
def kernel(x, table):
    flat = _gather(x.reshape(_BT).astype(jnp.int32), table)
    return flat.reshape(BATCH, HIST, DIM)


# speedup vs baseline: 1.0612x; 1.0066x over previous
import functools

import jax
import jax.numpy as jnp
from jax import lax
from jax.experimental import pallas as pl
from jax.experimental.pallas import tpu as pltpu
from jax.experimental.pallas import tpu_sc as plsc

VOCAB = 1000000
DIM = 64
BATCH = 16384
HIST = 200

_NC = 2
_NS = 16
_NW = _NC * _NS

_BT = BATCH * HIST
_ROWS_PER_W = BATCH // _NW
_R = 4
_CHUNK = _R * HIST
_STEPS = _ROWS_PER_W // _R
_PAIRS = _STEPS // 2


def _make_gather():
    mesh = plsc.VectorSubcoreMesh(core_axis_name="c", subcore_axis_name="s")

    @functools.partial(
        pl.kernel,
        mesh=mesh,
        out_type=jax.ShapeDtypeStruct((BATCH, HIST, DIM), jnp.float32),
        scratch_types=[
            pltpu.VMEM((_CHUNK,), jnp.int32),
            pltpu.VMEM((_CHUNK,), jnp.int32),
            pltpu.VMEM((_CHUNK, DIM), jnp.float32),
            pltpu.VMEM((_CHUNK, DIM), jnp.float32),
            pltpu.SemaphoreType.DMA,
            pltpu.SemaphoreType.DMA,
            pltpu.SemaphoreType.DMA,
            pltpu.SemaphoreType.DMA,
        ],
        compiler_params=pltpu.CompilerParams(use_tc_tiling_on_sc=False),
    )
    def gather_kernel(idx_hbm, table_hbm, out_hbm,
                      idx0, idx1, rows0, rows1,
                      semg0, semg1, sems0, sems1):
        wid = lax.axis_index("s") * _NC + lax.axis_index("c")
        row_base = wid * _ROWS_PER_W
        idx = (idx0, idx1)
        rows = (rows0, rows1)
        semg = (semg0, semg1)
        sems = (sems0, sems1)

        def issue_gather(c, b):
            off = (row_base + c * _R) * HIST
            pltpu.sync_copy(idx_hbm.at[pl.ds(off, _CHUNK)], idx[b])
            pltpu.async_copy(table_hbm.at[idx[b]], rows[b], semg[b])

        def issue_stores(c, b):
            r0 = row_base + c * _R
            for k in range(_R):
                pltpu.async_copy(rows[b].at[pl.ds(k * HIST, HIST)],
                                 out_hbm.at[r0 + k], sems[b])

        def wait_stores(c, b):
            r0 = row_base + c * _R
            for k in range(_R):
                pltpu.make_async_copy(rows[b].at[pl.ds(k * HIST, HIST)],
                                      out_hbm.at[r0 + k], sems[b]).wait()

        issue_gather(0, 0)
        issue_gather(1, 1)

        def pair(p, carry):
            for b in range(2):
                c = 2 * p + b
                pltpu.make_async_copy(table_hbm.at[idx[b]], rows[b],
                                      semg[b]).wait()
                issue_stores(c, b)

                @pl.when(p < _PAIRS - 1)
                def _():
                    wait_stores(c, b)
                    issue_gather(c + 2, b)

            return carry

        lax.fori_loop(0, _PAIRS, pair, 0)

        wait_stores(_STEPS - 2, 0)
        wait_stores(_STEPS - 1, 1)

    return gather_kernel


_gather = _make_gather()


def kernel(x, table):
    return _gather(x.reshape(_BT).astype(jnp.int32), table)


# speedup vs baseline: 1.3524x; 1.2743x over previous
import functools

import jax
import jax.numpy as jnp
from jax import lax
from jax.experimental import pallas as pl
from jax.experimental.pallas import tpu as pltpu
from jax.experimental.pallas import tpu_sc as plsc
from jax.experimental.layout import Layout, with_layout_constraint

VOCAB = 1000000
DIM = 64
BATCH = 16384
HIST = 200

_NC = 2
_NS = 16
_NW = _NC * _NS

_BT = BATCH * HIST
_ROWS_PER_W = BATCH // _NW
_R = 4
_CHUNK = _R * HIST
_STEPS = _ROWS_PER_W // _R
_PAIRS = _STEPS // 2


def _make_gather():
    mesh = plsc.VectorSubcoreMesh(core_axis_name="c", subcore_axis_name="s")

    @functools.partial(
        pl.kernel,
        mesh=mesh,
        out_type=jax.ShapeDtypeStruct((BATCH, HIST, DIM), jnp.float32),
        scratch_types=[
            pltpu.VMEM((_CHUNK,), jnp.int32),
            pltpu.VMEM((_CHUNK,), jnp.int32),
            pltpu.VMEM((_CHUNK, DIM), jnp.float32),
            pltpu.VMEM((_CHUNK, DIM), jnp.float32),
            pltpu.SemaphoreType.DMA,
            pltpu.SemaphoreType.DMA,
            pltpu.SemaphoreType.DMA,
            pltpu.SemaphoreType.DMA,
        ],
        compiler_params=pltpu.CompilerParams(use_tc_tiling_on_sc=False),
    )
    def gather_kernel(idx_hbm, table_hbm, out_hbm,
                      idx0, idx1, rows0, rows1,
                      semg0, semg1, sems0, sems1):
        wid = lax.axis_index("s") * _NC + lax.axis_index("c")
        row_base = wid * _ROWS_PER_W
        idx = (idx0, idx1)
        rows = (rows0, rows1)
        semg = (semg0, semg1)
        sems = (sems0, sems1)

        def issue_gather(c, b):
            off = (row_base + c * _R) * HIST
            pltpu.sync_copy(idx_hbm.at[pl.ds(off, _CHUNK)], idx[b])
            pltpu.async_copy(table_hbm.at[idx[b]], rows[b], semg[b])

        def issue_stores(c, b):
            r0 = row_base + c * _R
            for k in range(_R):
                pltpu.async_copy(rows[b].at[pl.ds(k * HIST, HIST)],
                                 out_hbm.at[r0 + k], sems[b])

        def wait_stores(c, b):
            r0 = row_base + c * _R
            for k in range(_R):
                pltpu.make_async_copy(rows[b].at[pl.ds(k * HIST, HIST)],
                                      out_hbm.at[r0 + k], sems[b]).wait()

        issue_gather(0, 0)
        issue_gather(1, 1)

        def pair(p, carry):
            for b in range(2):
                c = 2 * p + b
                pltpu.make_async_copy(table_hbm.at[idx[b]], rows[b],
                                      semg[b]).wait()
                issue_stores(c, b)

                @pl.when(p < _PAIRS - 1)
                def _():
                    wait_stores(c, b)
                    issue_gather(c + 2, b)

            return carry

        lax.fori_loop(0, _PAIRS, pair, 0)

        wait_stores(_STEPS - 2, 0)
        wait_stores(_STEPS - 1, 1)

    return gather_kernel


_gather = _make_gather()


def kernel(x, table):
    out = _gather(x.reshape(_BT).astype(jnp.int32), table)
    return with_layout_constraint(out, Layout((0, 1, 2), tiling=()))
